# Initial kernel scaffold; baseline (speedup 1.0000x reference)
import jax
import jax.numpy as jnp
from jax.experimental import pallas as pl


def kernel(inp, W0, W1, W2, W3, b0, b1, b2, b3, knn0, knn1, knn2, knn3, fcW, fcb):
    raise NotImplementedError("write your pallas kernel here")


# speedup vs baseline: 2.6275x; 2.6275x over previous
import functools

import jax
import jax.numpy as jnp
from jax import lax
from jax.experimental import pallas as pl
from jax.experimental.pallas import tpu as pltpu
from jax.experimental.pallas import tpu_sc as plsc

_NC = 2
_NS = 16
_NW = _NC * _NS

_LAYER_CFG = [(5000, 160), (2500, 80), (1250, 40), (625, 24)]


def _lcn_layer(xT, knnf, wf, bp, cpu):
    dim_p = cpu * _NW
    n_groups = cpu // 2
    mesh = plsc.VectorSubcoreMesh(core_axis_name="c", subcore_axis_name="s")

    @functools.partial(
        pl.kernel,
        mesh=mesh,
        compiler_params=pltpu.CompilerParams(use_tc_tiling_on_sc=False),
        out_type=jax.ShapeDtypeStruct((dim_p, 32), jnp.float32),
        scratch_types=[
            pltpu.VMEM((128,), jnp.int32),
            pltpu.VMEM((128, 32), jnp.float32),
            pltpu.VMEM((128,), jnp.float32),
            pltpu.VMEM((cpu + 16,), jnp.float32),
            pltpu.VMEM((cpu, 32), jnp.float32),
            pltpu.SemaphoreType.DMA,
        ],
    )
    def body(xT_h, knn_h, w_h, b_h, out_h, idx_v, rows_v, w_v, b_v, out_v, sem):
        wid = lax.axis_index("s") * _NC + lax.axis_index("c")
        row0 = wid * cpu
        pltpu.sync_copy(b_h.at[pl.ds(row0, cpu)], b_v.at[pl.ds(0, cpu)])

        def group(g, carry):
            base = (row0 + g * 2) * 64
            pltpu.sync_copy(knn_h.at[pl.ds(base, 128)], idx_v)
            pltpu.async_copy(xT_h.at[idx_v], rows_v, sem).wait()
            pltpu.sync_copy(w_h.at[pl.ds(base, 128)], w_v)
            bv = b_v[pl.ds(g * 2, 16)]
            for u in range(2):
                l = g * 2 + u
                bb = bv[u]
                a0 = jnp.full((16,), bb, jnp.float32)
                a1 = jnp.full((16,), bb, jnp.float32)
                c0 = jnp.zeros((16,), jnp.float32)
                c1 = jnp.zeros((16,), jnp.float32)
                for q in range(4):
                    wq = w_v[pl.ds(u * 64 + q * 16, 16)]
                    for kk in range(0, 16, 2):
                        r = u * 64 + q * 16 + kk
                        w0 = wq[kk]
                        w1 = wq[kk + 1]
                        a0 = a0 + w0 * rows_v[r, 0:16]
                        a1 = a1 + w0 * rows_v[r, 16:32]
                        c0 = c0 + w1 * rows_v[r + 1, 0:16]
                        c1 = c1 + w1 * rows_v[r + 1, 16:32]
                out_v[l, 0:16] = a0 + c0
                out_v[l, 16:32] = a1 + c1
            return carry

        lax.fori_loop(0, n_groups, group, 0)
        pltpu.sync_copy(out_v, out_h.at[pl.ds(row0, cpu)])

    return body(xT, knnf, wf, bp)


def _fc_body(w_ref, x_ref, b_ref, o_ref):
    o_ref[...] = (
        jnp.dot(w_ref[...], x_ref[...], preferred_element_type=jnp.float32)
        + b_ref[...]
    )


def kernel(inp, W0, W1, W2, W3, b0, b1, b2, b3, knn0, knn1, knn2, knn3, fcW, fcb):
    Ws = [W0, W1, W2, W3]
    bs = [b0, b1, b2, b3]
    knns = [knn0, knn1, knn2, knn3]

    xT = inp[:, -1, :].T
    for i, (dim, cpu) in enumerate(_LAYER_CFG):
        dim_p = cpu * _NW
        pad = dim_p - dim
        knnf = jnp.pad(knns[i], ((0, pad), (0, 0))).reshape(-1)
        wf = jnp.pad(Ws[i], ((0, pad), (0, 0))).reshape(-1)
        bpad = jnp.pad(bs[i].reshape(-1), (0, pad))
        xT = _lcn_layer(xT, knnf, wf, bpad, cpu)

    d3p = xT.shape[0]
    fcWp = jnp.pad(fcW, ((0, 5), (0, d3p - fcW.shape[1])))
    fcbp = jnp.pad(fcb, (0, 5)).reshape(8, 1)
    angleT = pl.pallas_call(
        _fc_body,
        out_shape=jax.ShapeDtypeStruct((8, 32), jnp.float32),
    )(fcWp, xT, fcbp)
    return angleT[:3].T
